# baseline (speedup 1.0000x reference)
import dataclasses
import functools

import jax
import jax.numpy as jnp
from jax import lax
from jax.experimental import pallas as pl
from jax.experimental.pallas import tpu as pltpu
from jax.experimental.pallas import tpu_sc as plsc

N = 10000
D = 128
BLK = 128
NC, NS = 2, 16
NW = NC * NS
BPW = 80
NBLK = NW * BPW
E_PAD = NBLK * BLK
ROWS = 10112
RPS = ROWS // NS
DUMMY = N
TBLK = 32
NBUF = 5
TPW = E_PAD // (NW * TBLK)
NTR = E_PAD // TBLK
PHASES = 8
HPW = TPW // PHASES
EPW = TPW * TBLK
EPP = HPW * TBLK

_mesh = plsc.VectorSubcoreMesh(
    core_axis_name="c", subcore_axis_name="s", num_cores=NC, num_subcores=NS
)

_sc_params = pltpu.CompilerParams()
if "needs_layout_passes" in pltpu.CompilerParams.__dataclass_fields__:
    _sc_params = dataclasses.replace(_sc_params, needs_layout_passes=False)


@functools.partial(
    pl.kernel,
    out_type=jax.ShapeDtypeStruct((NW, ROWS), jnp.float32),
    mesh=_mesh,
    scratch_types=[
        pltpu.VMEM((ROWS,), jnp.float32),
        pltpu.VMEM((EPW,), jnp.int32),
    ],
    compiler_params=_sc_params,
)
def _sc_degree(dst_hbm, out_hbm, hist_v, idx_v):
    c = lax.axis_index("c")
    s = lax.axis_index("s")
    w = c * NS + s

    @pl.loop(0, ROWS, step=16)
    def _zero(i):
        hist_v[pl.ds(i, 16)] = jnp.zeros((16,), jnp.float32)

    pltpu.sync_copy(dst_hbm.at[pl.ds(w * EPW, EPW)], idx_v)
    ones = jnp.ones((16,), jnp.float32)

    @pl.loop(0, EPW, step=16)
    def _grp(k):
        idx = idx_v[pl.ds(k, 16)]
        plsc.addupdate_scatter(hist_v, [idx], ones)

    pltpu.sync_copy(hist_v, out_hbm.at[w])


@functools.partial(
    pl.kernel,
    out_type=jax.ShapeDtypeStruct((NC, ROWS, D), jnp.float32),
    mesh=_mesh,
    scratch_types=[
        pltpu.VMEM((2 * EPP,), jnp.int32),
        pltpu.VMEM((HPW, TBLK), jnp.int32),
        pltpu.VMEM((NBUF * TBLK, D), jnp.float32),
        pltpu.VMEM_SHARED((ROWS, D), jnp.float32),
        pltpu.SemaphoreType.DMA,
    ],
)
def _sc_aggregate(g_hbm, src_hbm, dst_hbm, out_hbm,
                  idx_v, di_v, rows_v, acc_sh, sem):
    c = lax.axis_index("c")
    s = lax.axis_index("s")
    w = c * NS + s
    bufs = tuple(rows_v.at[pl.ds(i * TBLK, TBLK)] for i in range(NBUF))

    @pl.loop(0, NBUF * TBLK)
    def _z0(i):
        @pl.loop(0, D, step=16)
        def _z1(k):
            rows_v[i, pl.ds(k, 16)] = jnp.zeros((16,), jnp.float32)

    zmain = RPS - RPS % (NBUF * TBLK)
    ztail = RPS - zmain

    @pl.loop(0, zmain, step=NBUF * TBLK)
    def _z2(r):
        pltpu.sync_copy(rows_v, acc_sh.at[pl.ds(s * RPS + r, NBUF * TBLK)])

    if ztail:
        pltpu.sync_copy(
            rows_v.at[pl.ds(0, ztail)],
            acc_sh.at[pl.ds(s * RPS + zmain, ztail)],
        )

    plsc.subcore_barrier()

    def _gref(t):
        return idx_v.at[pl.ds(t * TBLK, TBLK)]

    for ph in range(PHASES):
        base = (w * TPW + ph * HPW) * TBLK
        pltpu.sync_copy(src_hbm.at[pl.ds(base, EPP)], idx_v.at[pl.ds(0, EPP)])
        pltpu.sync_copy(dst_hbm.at[pl.ds(base, EPP)], idx_v.at[pl.ds(EPP, EPP)])

        @pl.loop(0, HPW)
        def _re(t):
            di_v[t, pl.ds(0, 16)] = idx_v[pl.ds(EPP + t * TBLK, 16)]
            di_v[t, pl.ds(16, 16)] = idx_v[pl.ds(EPP + t * TBLK + 16, 16)]

        for t in range(NBUF - 1):
            pltpu.async_copy(g_hbm.at[_gref(t)], bufs[t], sem)

        @pl.loop(0, HPW - NBUF, step=NBUF)
        def _edge(j):
            for i in range(NBUF):
                buf = bufs[i]
                nbuf = bufs[(i + NBUF - 1) % NBUF]
                t = j + i
                pltpu.make_async_copy(g_hbm.at[_gref(t)], buf, sem).wait()
                pltpu.async_copy(g_hbm.at[_gref(t + NBUF - 1)], nbuf, sem)
                pltpu.sync_copy(buf, acc_sh.at[di_v.at[t]], add=True)

        for i in range(NBUF):
            buf = bufs[i]
            nbuf = bufs[(i + NBUF - 1) % NBUF]
            t = HPW - NBUF + i
            pltpu.make_async_copy(g_hbm.at[_gref(t)], buf, sem).wait()
            if t + NBUF - 1 < HPW:
                pltpu.async_copy(g_hbm.at[_gref(t + NBUF - 1)], nbuf, sem)
            pltpu.sync_copy(buf, acc_sh.at[di_v.at[t]], add=True)

    plsc.subcore_barrier()

    pltpu.sync_copy(
        acc_sh.at[pl.ds(s * RPS, RPS)],
        out_hbm.at[c, pl.ds(s * RPS, RPS)],
    )


def _mm_body(x_ref, w_ref, o_ref):
    o_ref[...] = lax.dot_general(
        x_ref[...], w_ref[...], (((1,), (1,)), ((), ())),
        preferred_element_type=jnp.float32,
        precision=lax.Precision.HIGHEST,
    )


def _tc_linear(x, W):
    R = 2000
    return pl.pallas_call(
        _mm_body,
        grid=(N // R,),
        in_specs=[
            pl.BlockSpec((R, D), lambda i: (i, 0)),
            pl.BlockSpec((D, D), lambda i: (0, 0)),
        ],
        out_specs=pl.BlockSpec((R, D), lambda i: (i, 0)),
        out_shape=jax.ShapeDtypeStruct((N, D), jnp.float32),
    )(x, W)


def _scale_body(hist_ref, h_ref, g_ref, d_ref):
    ones = jnp.ones((NW, 1), jnp.float32)
    deg = lax.dot_general(
        hist_ref[...], ones, (((0,), (0,)), ((), ())),
        preferred_element_type=jnp.float32,
        precision=lax.Precision.HIGHEST,
    ) + 1.0
    dis = lax.rsqrt(deg[:N])
    d_ref[...] = dis
    g_ref[...] = h_ref[...] * dis


def _tc_scale(hist, h):
    return pl.pallas_call(
        _scale_body,
        out_shape=[
            jax.ShapeDtypeStruct((N, D), jnp.float32),
            jax.ShapeDtypeStruct((N, 1), jnp.float32),
        ],
    )(hist, h)


def _epi_body(p_ref0, p_ref1, g_ref, d_ref, b_ref, o_ref):
    p0 = p_ref0[0]
    p1 = p_ref1[0]
    o_ref[...] = d_ref[...] * (p0 + p1 + g_ref[...]) + b_ref[...]


def _tc_epilogue(parts, g, dis_col, b_row):
    R = 2000
    return pl.pallas_call(
        _epi_body,
        grid=(N // R,),
        in_specs=[
            pl.BlockSpec((1, R, D), lambda i: (0, i, 0)),
            pl.BlockSpec((1, R, D), lambda i: (1, i, 0)),
            pl.BlockSpec((R, D), lambda i: (i, 0)),
            pl.BlockSpec((R, 1), lambda i: (i, 0)),
            pl.BlockSpec((1, D), lambda i: (0, 0)),
        ],
        out_specs=pl.BlockSpec((R, D), lambda i: (i, 0)),
        out_shape=jax.ShapeDtypeStruct((N, D), jnp.float32),
    )(parts, parts, g, dis_col, b_row)


def kernel(x, edge_index, W, b):
    e = edge_index.shape[1]
    src = edge_index[0].astype(jnp.int32)
    dst = edge_index[1].astype(jnp.int32)
    pad = E_PAD - e
    pad_ar = jnp.arange(pad, dtype=jnp.int32)
    src_p = jnp.concatenate([src, pad_ar % N])
    dst_p = jnp.concatenate([dst, DUMMY + pad_ar % (ROWS - N)])

    h = _tc_linear(x, W)
    hist = _sc_degree(dst_p)
    g, dis_col = _tc_scale(hist, h)
    parts = _sc_aggregate(g, src_p, dst_p)
    out = _tc_epilogue(parts, g, dis_col, b.reshape(1, D))
    return out


# speedup vs baseline: 1.0456x; 1.0456x over previous
import dataclasses
import functools

import jax
import jax.numpy as jnp
from jax import lax
from jax.experimental import pallas as pl
from jax.experimental.pallas import tpu as pltpu
from jax.experimental.pallas import tpu_sc as plsc

N = 10000
D = 128
BLK = 128
NC, NS = 2, 16
NW = NC * NS
BPW = 80
NBLK = NW * BPW
E_PAD = NBLK * BLK
ROWS = 10112
RPS = ROWS // NS
DUMMY = N
TBLK = 32
NBUF = 5
TPW = E_PAD // (NW * TBLK)
NTR = E_PAD // TBLK
PHASES = 4
HPW = TPW // PHASES
EPW = TPW * TBLK
EPP = HPW * TBLK

_mesh = plsc.VectorSubcoreMesh(
    core_axis_name="c", subcore_axis_name="s", num_cores=NC, num_subcores=NS
)

_sc_params = pltpu.CompilerParams()
if "needs_layout_passes" in pltpu.CompilerParams.__dataclass_fields__:
    _sc_params = dataclasses.replace(_sc_params, needs_layout_passes=False)


@functools.partial(
    pl.kernel,
    out_type=jax.ShapeDtypeStruct((NW, ROWS), jnp.float32),
    mesh=_mesh,
    scratch_types=[
        pltpu.VMEM((ROWS,), jnp.float32),
        pltpu.VMEM((EPW,), jnp.int32),
    ],
    compiler_params=_sc_params,
)
def _sc_degree(dst_hbm, out_hbm, hist_v, idx_v):
    c = lax.axis_index("c")
    s = lax.axis_index("s")
    w = c * NS + s

    @pl.loop(0, ROWS, step=16)
    def _zero(i):
        hist_v[pl.ds(i, 16)] = jnp.zeros((16,), jnp.float32)

    pltpu.sync_copy(dst_hbm.at[pl.ds(w * EPW, EPW)], idx_v)
    ones = jnp.ones((16,), jnp.float32)

    @pl.loop(0, EPW, step=16)
    def _grp(k):
        idx = idx_v[pl.ds(k, 16)]
        plsc.addupdate_scatter(hist_v, [idx], ones)

    pltpu.sync_copy(hist_v, out_hbm.at[w])


@functools.partial(
    pl.kernel,
    out_type=jax.ShapeDtypeStruct((NC, ROWS, D), jnp.float32),
    mesh=_mesh,
    scratch_types=[
        pltpu.VMEM((2 * EPP,), jnp.int32),
        pltpu.VMEM((HPW, TBLK), jnp.int32),
        pltpu.VMEM((NBUF * TBLK, D), jnp.float32),
        pltpu.VMEM_SHARED((ROWS, D), jnp.float32),
        pltpu.SemaphoreType.DMA,
    ],
)
def _sc_aggregate(g_hbm, src_hbm, dst_hbm, out_hbm,
                  idx_v, di_v, rows_v, acc_sh, sem):
    c = lax.axis_index("c")
    s = lax.axis_index("s")
    w = c * NS + s
    bufs = tuple(rows_v.at[pl.ds(i * TBLK, TBLK)] for i in range(NBUF))

    @pl.loop(0, NBUF * TBLK)
    def _z0(i):
        @pl.loop(0, D, step=16)
        def _z1(k):
            rows_v[i, pl.ds(k, 16)] = jnp.zeros((16,), jnp.float32)

    zmain = RPS - RPS % (NBUF * TBLK)
    ztail = RPS - zmain

    @pl.loop(0, zmain, step=NBUF * TBLK)
    def _z2(r):
        pltpu.sync_copy(rows_v, acc_sh.at[pl.ds(s * RPS + r, NBUF * TBLK)])

    if ztail:
        pltpu.sync_copy(
            rows_v.at[pl.ds(0, ztail)],
            acc_sh.at[pl.ds(s * RPS + zmain, ztail)],
        )

    plsc.subcore_barrier()

    def _gref(t):
        return idx_v.at[pl.ds(t * TBLK, TBLK)]

    for ph in range(PHASES):
        base = (w * TPW + ph * HPW) * TBLK
        pltpu.sync_copy(src_hbm.at[pl.ds(base, EPP)], idx_v.at[pl.ds(0, EPP)])
        pltpu.sync_copy(dst_hbm.at[pl.ds(base, EPP)], idx_v.at[pl.ds(EPP, EPP)])

        @pl.loop(0, HPW)
        def _re(t):
            di_v[t, pl.ds(0, 16)] = idx_v[pl.ds(EPP + t * TBLK, 16)]
            di_v[t, pl.ds(16, 16)] = idx_v[pl.ds(EPP + t * TBLK + 16, 16)]

        for t in range(NBUF - 1):
            pltpu.async_copy(g_hbm.at[_gref(t)], bufs[t], sem)

        @pl.loop(0, HPW - NBUF, step=NBUF)
        def _edge(j):
            for i in range(NBUF):
                buf = bufs[i]
                nbuf = bufs[(i + NBUF - 1) % NBUF]
                t = j + i
                pltpu.make_async_copy(g_hbm.at[_gref(t)], buf, sem).wait()
                pltpu.async_copy(g_hbm.at[_gref(t + NBUF - 1)], nbuf, sem)
                pltpu.sync_copy(buf, acc_sh.at[di_v.at[t]], add=True)

        for i in range(NBUF):
            buf = bufs[i]
            nbuf = bufs[(i + NBUF - 1) % NBUF]
            t = HPW - NBUF + i
            pltpu.make_async_copy(g_hbm.at[_gref(t)], buf, sem).wait()
            if t + NBUF - 1 < HPW:
                pltpu.async_copy(g_hbm.at[_gref(t + NBUF - 1)], nbuf, sem)
            pltpu.sync_copy(buf, acc_sh.at[di_v.at[t]], add=True)

    plsc.subcore_barrier()

    pltpu.sync_copy(
        acc_sh.at[pl.ds(s * RPS, RPS)],
        out_hbm.at[c, pl.ds(s * RPS, RPS)],
    )


def _mm_body(x_ref, w_ref, o_ref):
    o_ref[...] = lax.dot_general(
        x_ref[...], w_ref[...], (((1,), (1,)), ((), ())),
        preferred_element_type=jnp.float32,
        precision=lax.Precision.HIGHEST,
    )


def _tc_linear(x, W):
    R = 2000
    return pl.pallas_call(
        _mm_body,
        grid=(N // R,),
        in_specs=[
            pl.BlockSpec((R, D), lambda i: (i, 0)),
            pl.BlockSpec((D, D), lambda i: (0, 0)),
        ],
        out_specs=pl.BlockSpec((R, D), lambda i: (i, 0)),
        out_shape=jax.ShapeDtypeStruct((N, D), jnp.float32),
    )(x, W)


def _scale_body(hist_ref, h_ref, g_ref, d_ref):
    ones = jnp.ones((NW, 1), jnp.float32)
    deg = lax.dot_general(
        hist_ref[...], ones, (((0,), (0,)), ((), ())),
        preferred_element_type=jnp.float32,
        precision=lax.Precision.HIGHEST,
    ) + 1.0
    dis = lax.rsqrt(deg[:N])
    d_ref[...] = dis
    g_ref[...] = h_ref[...] * dis


def _tc_scale(hist, h):
    return pl.pallas_call(
        _scale_body,
        out_shape=[
            jax.ShapeDtypeStruct((N, D), jnp.float32),
            jax.ShapeDtypeStruct((N, 1), jnp.float32),
        ],
    )(hist, h)


def _epi_body(p_ref0, p_ref1, g_ref, d_ref, b_ref, o_ref):
    p0 = p_ref0[0]
    p1 = p_ref1[0]
    o_ref[...] = d_ref[...] * (p0 + p1 + g_ref[...]) + b_ref[...]


def _tc_epilogue(parts, g, dis_col, b_row):
    R = 2000
    return pl.pallas_call(
        _epi_body,
        grid=(N // R,),
        in_specs=[
            pl.BlockSpec((1, R, D), lambda i: (0, i, 0)),
            pl.BlockSpec((1, R, D), lambda i: (1, i, 0)),
            pl.BlockSpec((R, D), lambda i: (i, 0)),
            pl.BlockSpec((R, 1), lambda i: (i, 0)),
            pl.BlockSpec((1, D), lambda i: (0, 0)),
        ],
        out_specs=pl.BlockSpec((R, D), lambda i: (i, 0)),
        out_shape=jax.ShapeDtypeStruct((N, D), jnp.float32),
    )(parts, parts, g, dis_col, b_row)


def kernel(x, edge_index, W, b):
    e = edge_index.shape[1]
    src = edge_index[0].astype(jnp.int32)
    dst = edge_index[1].astype(jnp.int32)
    pad = E_PAD - e
    pad_ar = jnp.arange(pad, dtype=jnp.int32)
    src_p = jnp.concatenate([src, pad_ar % N])
    dst_p = jnp.concatenate([dst, DUMMY + pad_ar % (ROWS - N)])

    h = _tc_linear(x, W)
    hist = _sc_degree(dst_p)
    g, dis_col = _tc_scale(hist, h)
    parts = _sc_aggregate(g, src_p, dst_p)
    out = _tc_epilogue(parts, g, dis_col, b.reshape(1, D))
    return out


# speedup vs baseline: 1.0465x; 1.0009x over previous
import dataclasses
import functools

import jax
import jax.numpy as jnp
from jax import lax
from jax.experimental import pallas as pl
from jax.experimental.pallas import tpu as pltpu
from jax.experimental.pallas import tpu_sc as plsc

N = 10000
D = 128
BLK = 128
NC, NS = 2, 16
NW = NC * NS
BPW = 80
NBLK = NW * BPW
E_PAD = NBLK * BLK
ROWS = 10112
RPS = ROWS // NS
DUMMY = N
TBLK = 32
NBUF = 5
TPW = E_PAD // (NW * TBLK)
NTR = E_PAD // TBLK
PHASES = 4
HPW = TPW // PHASES
EPW = TPW * TBLK
EPP = HPW * TBLK

_mesh = plsc.VectorSubcoreMesh(
    core_axis_name="c", subcore_axis_name="s", num_cores=NC, num_subcores=NS
)

_sc_params = pltpu.CompilerParams()
if "needs_layout_passes" in pltpu.CompilerParams.__dataclass_fields__:
    _sc_params = dataclasses.replace(_sc_params, needs_layout_passes=False)


@functools.partial(
    pl.kernel,
    out_type=jax.ShapeDtypeStruct((NW, ROWS), jnp.float32),
    mesh=_mesh,
    scratch_types=[
        pltpu.VMEM((ROWS,), jnp.float32),
        pltpu.VMEM((EPW,), jnp.int32),
    ],
    compiler_params=_sc_params,
)
def _sc_degree(dst_hbm, out_hbm, hist_v, idx_v):
    c = lax.axis_index("c")
    s = lax.axis_index("s")
    w = c * NS + s

    @pl.loop(0, ROWS, step=16)
    def _zero(i):
        hist_v[pl.ds(i, 16)] = jnp.zeros((16,), jnp.float32)

    pltpu.sync_copy(dst_hbm.at[pl.ds(w * EPW, EPW)], idx_v)
    ones = jnp.ones((16,), jnp.float32)

    @pl.loop(0, EPW, step=16)
    def _grp(k):
        idx = idx_v[pl.ds(k, 16)]
        plsc.addupdate_scatter(hist_v, [idx], ones)

    pltpu.sync_copy(hist_v, out_hbm.at[w])


@functools.partial(
    pl.kernel,
    out_type=jax.ShapeDtypeStruct((NC, ROWS, D), jnp.float32),
    mesh=_mesh,
    scratch_types=[
        pltpu.VMEM((2 * EPP,), jnp.int32),
        pltpu.VMEM((HPW, TBLK), jnp.int32),
        pltpu.VMEM((NBUF * TBLK, D), jnp.float32),
        pltpu.VMEM_SHARED((ROWS, D), jnp.float32),
        pltpu.SemaphoreType.DMA,
    ],
)
def _sc_aggregate(g_hbm, src_hbm, dst_hbm, out_hbm,
                  idx_v, di_v, rows_v, acc_sh, sem):
    c = lax.axis_index("c")
    s = lax.axis_index("s")
    w = c * NS + s
    bufs = tuple(rows_v.at[pl.ds(i * TBLK, TBLK)] for i in range(NBUF))

    @pl.loop(0, NBUF * TBLK)
    def _z0(i):
        @pl.loop(0, D, step=16)
        def _z1(k):
            rows_v[i, pl.ds(k, 16)] = jnp.zeros((16,), jnp.float32)

    zstep = NBUF * TBLK
    zmain = RPS - RPS % zstep
    ztail = RPS - zmain

    @pl.loop(0, zmain, step=zstep)
    def _z2(r):
        pltpu.sync_copy(rows_v, acc_sh.at[pl.ds(s * RPS + r, zstep)])

    if ztail:
        pltpu.sync_copy(
            rows_v.at[pl.ds(0, ztail)],
            acc_sh.at[pl.ds(s * RPS + zmain, ztail)],
        )

    plsc.subcore_barrier()

    def _gref(t):
        return idx_v.at[pl.ds(t * TBLK, TBLK)]

    for ph in range(PHASES):
        base = (w * TPW + ph * HPW) * TBLK
        pltpu.sync_copy(src_hbm.at[pl.ds(base, EPP)], idx_v.at[pl.ds(0, EPP)])
        pltpu.sync_copy(dst_hbm.at[pl.ds(base, EPP)], idx_v.at[pl.ds(EPP, EPP)])

        @pl.loop(0, HPW)
        def _re(t):
            di_v[t, pl.ds(0, 16)] = idx_v[pl.ds(EPP + t * TBLK, 16)]
            di_v[t, pl.ds(16, 16)] = idx_v[pl.ds(EPP + t * TBLK + 16, 16)]

        for t in range(NBUF - 1):
            pltpu.async_copy(g_hbm.at[_gref(t)], bufs[t], sem)

        @pl.loop(0, HPW - NBUF, step=NBUF)
        def _edge(j):
            for i in range(NBUF):
                buf = bufs[i]
                nbuf = bufs[(i + NBUF - 1) % NBUF]
                t = j + i
                pltpu.make_async_copy(g_hbm.at[_gref(t)], buf, sem).wait()
                pltpu.async_copy(g_hbm.at[_gref(t + NBUF - 1)], nbuf, sem)
                pltpu.sync_copy(buf, acc_sh.at[di_v.at[t]], add=True)

        for i in range(NBUF):
            buf = bufs[i]
            nbuf = bufs[(i + NBUF - 1) % NBUF]
            t = HPW - NBUF + i
            pltpu.make_async_copy(g_hbm.at[_gref(t)], buf, sem).wait()
            if t + NBUF - 1 < HPW:
                pltpu.async_copy(g_hbm.at[_gref(t + NBUF - 1)], nbuf, sem)
            pltpu.sync_copy(buf, acc_sh.at[di_v.at[t]], add=True)

    plsc.subcore_barrier()

    pltpu.sync_copy(
        acc_sh.at[pl.ds(s * RPS, RPS)],
        out_hbm.at[c, pl.ds(s * RPS, RPS)],
    )


def _mm_body(x_ref, w_ref, o_ref):
    o_ref[...] = lax.dot_general(
        x_ref[...], w_ref[...], (((1,), (1,)), ((), ())),
        preferred_element_type=jnp.float32,
        precision=lax.Precision.HIGHEST,
    )


def _tc_linear(x, W):
    R = 2000
    return pl.pallas_call(
        _mm_body,
        grid=(N // R,),
        in_specs=[
            pl.BlockSpec((R, D), lambda i: (i, 0)),
            pl.BlockSpec((D, D), lambda i: (0, 0)),
        ],
        out_specs=pl.BlockSpec((R, D), lambda i: (i, 0)),
        out_shape=jax.ShapeDtypeStruct((N, D), jnp.float32),
    )(x, W)


def _scale_body(hist_ref, h_ref, g_ref, d_ref):
    ones = jnp.ones((NW, 1), jnp.float32)
    deg = lax.dot_general(
        hist_ref[...], ones, (((0,), (0,)), ((), ())),
        preferred_element_type=jnp.float32,
        precision=lax.Precision.HIGHEST,
    ) + 1.0
    dis = lax.rsqrt(deg[:N])
    d_ref[...] = dis
    g_ref[...] = h_ref[...] * dis


def _tc_scale(hist, h):
    return pl.pallas_call(
        _scale_body,
        out_shape=[
            jax.ShapeDtypeStruct((N, D), jnp.float32),
            jax.ShapeDtypeStruct((N, 1), jnp.float32),
        ],
    )(hist, h)


def _epi_body(p_ref0, p_ref1, g_ref, d_ref, b_ref, o_ref):
    p0 = p_ref0[0]
    p1 = p_ref1[0]
    o_ref[...] = d_ref[...] * (p0 + p1 + g_ref[...]) + b_ref[...]


def _tc_epilogue(parts, g, dis_col, b_row):
    R = 2000
    return pl.pallas_call(
        _epi_body,
        grid=(N // R,),
        in_specs=[
            pl.BlockSpec((1, R, D), lambda i: (0, i, 0)),
            pl.BlockSpec((1, R, D), lambda i: (1, i, 0)),
            pl.BlockSpec((R, D), lambda i: (i, 0)),
            pl.BlockSpec((R, 1), lambda i: (i, 0)),
            pl.BlockSpec((1, D), lambda i: (0, 0)),
        ],
        out_specs=pl.BlockSpec((R, D), lambda i: (i, 0)),
        out_shape=jax.ShapeDtypeStruct((N, D), jnp.float32),
    )(parts, parts, g, dis_col, b_row)


def kernel(x, edge_index, W, b):
    e = edge_index.shape[1]
    src = edge_index[0].astype(jnp.int32)
    dst = edge_index[1].astype(jnp.int32)
    pad = E_PAD - e
    pad_ar = jnp.arange(pad, dtype=jnp.int32)
    src_p = jnp.concatenate([src, pad_ar % N])
    dst_p = jnp.concatenate([dst, DUMMY + pad_ar % (ROWS - N)])

    h = _tc_linear(x, W)
    hist = _sc_degree(dst_p)
    g, dis_col = _tc_scale(hist, h)
    parts = _sc_aggregate(g, src_p, dst_p)
    out = _tc_epilogue(parts, g, dis_col, b.reshape(1, D))
    return out
